# baseline (speedup 1.0000x reference)
import jax
import jax.numpy as jnp
from jax import lax
from jax.experimental import pallas as pl
from jax.experimental.pallas import tpu as pltpu
from jax.experimental.pallas import tpu_sc as plsc

N_NODES = 100000
N_EDGES = 6400000
ROW = 128
NROWS = N_EDGES // ROW
K = 8
NGROUPS = NROWS // K
NC = 2
NS = 16
NW = NC * NS
GE = K * ROW
NPAD = 102400
NPT = NPAD // NS


def _sc_body(*refs):
    (ht_hbm, src_hbm, dst_hbm, z_hbm, out_hbm,
     sidx0, sidx1, didx0, didx1,
     a_hs0, a_hs1, a_hs2, a_hs3, a_hd0, a_hd1, a_hd2, a_hd3,
     b_hs0, b_hs1, b_hs2, b_hs3, b_hd0, b_hd1, b_hd2, b_hd3,
     mg0, mg1, mg2, mg3, mg4, stage,
     sh0, sh1, sh2, sh3, ac0, ac1, ac2, ac3, ac4,
     sem_i, sem_g0, sem_g1, sem_s) = refs
    sidx = [sidx0, sidx1]
    didx = [didx0, didx1]
    hs = [[a_hs0, a_hs1, a_hs2, a_hs3], [b_hs0, b_hs1, b_hs2, b_hs3]]
    hd = [[a_hd0, a_hd1, a_hd2, a_hd3], [b_hd0, b_hd1, b_hd2, b_hd3]]
    mg = [mg0, mg1, mg2, mg3, mg4]
    hsh = [sh0, sh1, sh2, sh3]
    acc = [ac0, ac1, ac2, ac3, ac4]
    semg = [sem_g0, sem_g1]

    c = lax.axis_index("c")
    s = lax.axis_index("s")
    wid = s * NC + c
    row0 = s * NPT

    for i in range(4):
        @pl.when(s == i)
        def _(i=i):
            pltpu.sync_copy(ht_hbm.at[pl.ds(i * NPAD, NPAD)], hsh[i])

    pltpu.sync_copy(z_hbm.at[pl.ds(row0, NPT)], stage)
    for comp in range(5):
        pltpu.sync_copy(stage, acc[comp].at[pl.ds(row0, NPT)])

    ones = jnp.full((16,), 1.0, jnp.float32)
    for i in range(GE // 16):
        mg4[pl.ds(i * 16, 16)] = ones

    plsc.subcore_barrier()

    n_g = jnp.where(wid < NGROUPS % NW, NGROUPS // NW + 1, NGROUPS // NW)

    def gather_descs(slot):
        ds_ = []
        for j in range(K):
            dsl = pl.ds(j * ROW, ROW)
            for comp in range(4):
                ds_.append((hsh[comp].at[sidx[slot].at[j]],
                            hs[slot][comp].at[dsl], semg[slot]))
                ds_.append((hsh[comp].at[didx[slot].at[j]],
                            hd[slot][comp].at[dsl], semg[slot]))
        return ds_

    def load_and_fire(slot, t):
        g = wid + t * NW
        r0 = g * K
        cp1 = pltpu.async_copy(src_hbm.at[pl.ds(r0, K)], sidx[slot], sem_i)
        cp2 = pltpu.async_copy(dst_hbm.at[pl.ds(r0, K)], didx[slot], sem_i)
        cp1.wait()
        cp2.wait()
        for src, dst, sem in gather_descs(slot):
            pltpu.async_copy(src, dst, sem)

    def wait_gathers(slot):
        for src, dst, sem in gather_descs(slot):
            pltpu.make_async_copy(src, dst, sem).wait()

    def compute_and_scatter(slot):
        for j in range(K):
            for v in range(ROW // 16):
                vsl = pl.ds(j * ROW + v * 16, 16)

                def ld(ref):
                    return ref[vsl]

                a0 = ld(hd[slot][0])
                a1 = ld(hd[slot][1])
                a2 = ld(hd[slot][2])
                a3 = ld(hd[slot][3])
                b0 = ld(hs[slot][0])
                b1 = ld(hs[slot][1])
                b2 = ld(hs[slot][2])
                b3 = ld(hs[slot][3])
                x0 = a0 - b0
                x1 = a1 - b1
                t0 = x0 * 0.07104663
                m0 = (x0 - x1 / (t0 * t0 + 1.536996)) * -0.028956918
                t1 = x0 * -0.021992652
                m1 = (x0 - x1 * (0.8290067 - t1 * t1)) * 0.025425926
                t2 = x0 * -0.083299406
                m2 = (x0 - t2 * t2) * -0.024002103 - 0.22298379
                m3 = (x1 + 2.6200492 + x0 * -0.16023761) * 0.025031794
                zm = (x0 == 0.0) & (x1 == 0.0) & (a2 == b2) & (a3 == b3)
                mg0[vsl] = jnp.where(zm, 0.0, m0)
                mg1[vsl] = jnp.where(zm, 0.0, m1)
                mg2[vsl] = jnp.where(zm, 0.0, m2)
                mg3[vsl] = jnp.where(zm, 0.0, m3)
        pend = []
        for j in range(K):
            dsl = pl.ds(j * ROW, ROW)
            for comp in range(5):
                pend.append(pltpu.async_copy(
                    mg[comp].at[dsl], acc[comp].at[didx[slot].at[j]],
                    sem_s, add=True))
        for p in pend:
            p.wait()

    def stage_fn(slot, t):
        wait_gathers(slot)
        t1 = t + 1

        @pl.when(t1 < n_g)
        def _():
            load_and_fire(1 - slot, t1)

        compute_and_scatter(slot)

    load_and_fire(0, 0)

    def pair_body(p, carry):
        stage_fn(0, 2 * p)
        stage_fn(1, 2 * p + 1)
        return carry

    lax.fori_loop(0, n_g // 2, pair_body, 0)

    @pl.when(n_g % 2 == 1)
    def _():
        stage_fn(0, n_g - 1)

    plsc.subcore_barrier()
    for comp in range(5):
        pltpu.sync_copy(acc[comp].at[pl.ds(row0, NPT)], stage)
        obase = (c * 5 + comp) * NPAD + row0
        pltpu.sync_copy(stage, out_hbm.at[pl.ds(obase, NPT)])


def _decode_body(p_ref, o_ref):
    p = p_ref[...]
    a = p[0] + p[1]
    y0 = a[2:3, :]
    y1 = a[3:4, :]
    cnt = jnp.maximum(a[4:5, :], 1.0)
    y2 = a[0:1, :] / cnt
    y3 = a[1:2, :] / cnt
    t0 = y2 * 0.15994334
    u0 = (y0 - (y3 + t0 * t0) / 1.7044706 - y2) * 0.16596459
    t1 = y2 * -0.089175865
    u1 = (y1 - t1 * t1 * y3 - y2 + y3) * -0.05459863
    u2 = (y3 + y0) * 0.05392959
    u3 = y2 * (12.305774 / (y2 * y2 + 63.129406))
    p0 = ((u0 / 0.5268826 + u3 - u2) * -0.18549965 - (u1 + u2)) / 0.7328953
    p1 = u0 * -0.8037861 - u1 + (u3 * 1.2175907 + u2)
    o_ref[...] = jnp.concatenate([p0, p1], axis=0)


def kernel(pos, vel, edge_index):
    h = jnp.concatenate([pos.astype(jnp.float32), vel.astype(jnp.float32)],
                        axis=1)
    ht = jnp.zeros((4, NPAD), jnp.float32).at[:, :N_NODES].set(h.T).reshape(-1)
    ei = edge_index.astype(jnp.int32)
    src = ei[0].reshape(NROWS, ROW)
    dst = ei[1].reshape(NROWS, ROW)
    zeros = jnp.zeros((NPAD,), jnp.float32)

    mesh = plsc.VectorSubcoreMesh(core_axis_name="c", subcore_axis_name="s")
    plane_f32 = pltpu.VMEM_SHARED((NPAD,), jnp.float32)
    buf_f32 = pltpu.VMEM((GE,), jnp.float32)
    sc = pl.kernel(
        _sc_body,
        out_type=jax.ShapeDtypeStruct((NC * 5 * NPAD,), jnp.float32),
        mesh=mesh,
        scratch_types=(
            [pltpu.VMEM((K, ROW), jnp.int32)] * 4
            + [buf_f32] * 8
            + [buf_f32] * 8
            + [buf_f32] * 5
            + [pltpu.VMEM((NPT,), jnp.float32)]
            + [plane_f32] * 4
            + [plane_f32] * 5
            + [pltpu.SemaphoreType.DMA] * 4
        ),
    )
    parts = sc(ht, src, dst, zeros).reshape(NC, 5, NPAD)

    bn = NPAD // 8
    dec = pl.pallas_call(
        _decode_body,
        grid=(NPAD // bn,),
        in_specs=[pl.BlockSpec((NC, 5, bn), lambda i: (0, 0, i))],
        out_specs=pl.BlockSpec((2, bn), lambda i: (0, i)),
        out_shape=jax.ShapeDtypeStruct((2, NPAD), jnp.float32),
    )
    return dec(parts)[:, :N_NODES].T


# speedup vs baseline: 1.2687x; 1.2687x over previous
import jax
import jax.numpy as jnp
from jax import lax
from jax.experimental import pallas as pl
from jax.experimental.pallas import tpu as pltpu
from jax.experimental.pallas import tpu_sc as plsc

N_NODES = 100000
N_EDGES = 6400000
ROW = 128
NROWS = N_EDGES // ROW
K = 16
NGROUPS = NROWS // K
NC = 2
NS = 16
NW = NC * NS
GE = K * ROW
NPAD = 102400
NPT = NPAD // NS


def _sc_body(*refs):
    (ht_hbm, src_hbm, dst_hbm, z_hbm, out_hbm,
     sidx0, sidx1, didx0, didx1,
     a_hs0, a_hs1, a_hs2, a_hs3, a_hd0, a_hd1, a_hd2, a_hd3,
     b_hs0, b_hs1, b_hs2, b_hs3, b_hd0, b_hd1, b_hd2, b_hd3,
     mg0, mg1, mg2, mg3, mg4, stage,
     sh0, sh1, sh2, sh3, ac0, ac1, ac2, ac3, ac4,
     sem_i, sem_g0, sem_g1, sem_s) = refs
    sidx = [sidx0, sidx1]
    didx = [didx0, didx1]
    hs = [[a_hs0, a_hs1, a_hs2, a_hs3], [b_hs0, b_hs1, b_hs2, b_hs3]]
    hd = [[a_hd0, a_hd1, a_hd2, a_hd3], [b_hd0, b_hd1, b_hd2, b_hd3]]
    mg = [mg0, mg1, mg2, mg3, mg4]
    hsh = [sh0, sh1, sh2, sh3]
    acc = [ac0, ac1, ac2, ac3, ac4]
    semg = [sem_g0, sem_g1]

    c = lax.axis_index("c")
    s = lax.axis_index("s")
    wid = s * NC + c
    row0 = s * NPT

    for i in range(4):
        @pl.when(s == i)
        def _(i=i):
            pltpu.sync_copy(ht_hbm.at[pl.ds(i * NPAD, NPAD)], hsh[i])

    pltpu.sync_copy(z_hbm.at[pl.ds(row0, NPT)], stage)
    for comp in range(5):
        pltpu.sync_copy(stage, acc[comp].at[pl.ds(row0, NPT)])

    ones = jnp.full((16,), 1.0, jnp.float32)
    for i in range(GE // 16):
        mg4[pl.ds(i * 16, 16)] = ones

    plsc.subcore_barrier()

    n_g = jnp.where(wid < NGROUPS % NW, NGROUPS // NW + 1, NGROUPS // NW)

    def gather_descs(slot):
        ds_ = []
        for j in range(K):
            dsl = pl.ds(j * ROW, ROW)
            for comp in range(4):
                ds_.append((hsh[comp].at[sidx[slot].at[j]],
                            hs[slot][comp].at[dsl], semg[slot]))
                ds_.append((hsh[comp].at[didx[slot].at[j]],
                            hd[slot][comp].at[dsl], semg[slot]))
        return ds_

    def load_and_fire(slot, t):
        g = wid + t * NW
        r0 = g * K
        cp1 = pltpu.async_copy(src_hbm.at[pl.ds(r0, K)], sidx[slot], sem_i)
        cp2 = pltpu.async_copy(dst_hbm.at[pl.ds(r0, K)], didx[slot], sem_i)
        cp1.wait()
        cp2.wait()
        for src, dst, sem in gather_descs(slot):
            pltpu.async_copy(src, dst, sem)

    def wait_gathers(slot):
        for src, dst, sem in gather_descs(slot):
            pltpu.make_async_copy(src, dst, sem).wait()

    def compute_and_scatter(slot):
        for j in range(K):
            for v in range(ROW // 16):
                vsl = pl.ds(j * ROW + v * 16, 16)

                def ld(ref):
                    return ref[vsl]

                a0 = ld(hd[slot][0])
                a1 = ld(hd[slot][1])
                a2 = ld(hd[slot][2])
                a3 = ld(hd[slot][3])
                b0 = ld(hs[slot][0])
                b1 = ld(hs[slot][1])
                b2 = ld(hs[slot][2])
                b3 = ld(hs[slot][3])
                x0 = a0 - b0
                x1 = a1 - b1
                t0 = x0 * 0.07104663
                m0 = (x0 - x1 / (t0 * t0 + 1.536996)) * -0.028956918
                t1 = x0 * -0.021992652
                m1 = (x0 - x1 * (0.8290067 - t1 * t1)) * 0.025425926
                t2 = x0 * -0.083299406
                m2 = (x0 - t2 * t2) * -0.024002103 - 0.22298379
                m3 = (x1 + 2.6200492 + x0 * -0.16023761) * 0.025031794
                zm = (x0 == 0.0) & (x1 == 0.0) & (a2 == b2) & (a3 == b3)
                mg0[vsl] = jnp.where(zm, 0.0, m0)
                mg1[vsl] = jnp.where(zm, 0.0, m1)
                mg2[vsl] = jnp.where(zm, 0.0, m2)
                mg3[vsl] = jnp.where(zm, 0.0, m3)
        pend = []
        for j in range(K):
            dsl = pl.ds(j * ROW, ROW)
            for comp in range(5):
                pend.append(pltpu.async_copy(
                    mg[comp].at[dsl], acc[comp].at[didx[slot].at[j]],
                    sem_s, add=True))
        for p in pend:
            p.wait()

    def stage_fn(slot, t):
        wait_gathers(slot)
        t1 = t + 1

        @pl.when(t1 < n_g)
        def _():
            load_and_fire(1 - slot, t1)

        compute_and_scatter(slot)

    load_and_fire(0, 0)

    def pair_body(p, carry):
        stage_fn(0, 2 * p)
        stage_fn(1, 2 * p + 1)
        return carry

    lax.fori_loop(0, n_g // 2, pair_body, 0)

    @pl.when(n_g % 2 == 1)
    def _():
        stage_fn(0, n_g - 1)

    plsc.subcore_barrier()
    for comp in range(5):
        pltpu.sync_copy(acc[comp].at[pl.ds(row0, NPT)], stage)
        obase = (c * 5 + comp) * NPAD + row0
        pltpu.sync_copy(stage, out_hbm.at[pl.ds(obase, NPT)])


def _decode_body(p_ref, o_ref):
    p = p_ref[...]
    a = p[0] + p[1]
    y0 = a[2:3, :]
    y1 = a[3:4, :]
    cnt = jnp.maximum(a[4:5, :], 1.0)
    y2 = a[0:1, :] / cnt
    y3 = a[1:2, :] / cnt
    t0 = y2 * 0.15994334
    u0 = (y0 - (y3 + t0 * t0) / 1.7044706 - y2) * 0.16596459
    t1 = y2 * -0.089175865
    u1 = (y1 - t1 * t1 * y3 - y2 + y3) * -0.05459863
    u2 = (y3 + y0) * 0.05392959
    u3 = y2 * (12.305774 / (y2 * y2 + 63.129406))
    p0 = ((u0 / 0.5268826 + u3 - u2) * -0.18549965 - (u1 + u2)) / 0.7328953
    p1 = u0 * -0.8037861 - u1 + (u3 * 1.2175907 + u2)
    o_ref[...] = jnp.concatenate([p0, p1], axis=0)


def kernel(pos, vel, edge_index):
    h = jnp.concatenate([pos.astype(jnp.float32), vel.astype(jnp.float32)],
                        axis=1)
    ht = jnp.zeros((4, NPAD), jnp.float32).at[:, :N_NODES].set(h.T).reshape(-1)
    ei = edge_index.astype(jnp.int32)
    src = ei[0].reshape(NROWS, ROW)
    dst = ei[1].reshape(NROWS, ROW)
    zeros = jnp.zeros((NPAD,), jnp.float32)

    mesh = plsc.VectorSubcoreMesh(core_axis_name="c", subcore_axis_name="s")
    plane_f32 = pltpu.VMEM_SHARED((NPAD,), jnp.float32)
    buf_f32 = pltpu.VMEM((GE,), jnp.float32)
    sc = pl.kernel(
        _sc_body,
        out_type=jax.ShapeDtypeStruct((NC * 5 * NPAD,), jnp.float32),
        mesh=mesh,
        scratch_types=(
            [pltpu.VMEM((K, ROW), jnp.int32)] * 4
            + [buf_f32] * 8
            + [buf_f32] * 8
            + [buf_f32] * 5
            + [pltpu.VMEM((NPT,), jnp.float32)]
            + [plane_f32] * 4
            + [plane_f32] * 5
            + [pltpu.SemaphoreType.DMA] * 4
        ),
    )
    parts = sc(ht, src, dst, zeros).reshape(NC, 5, NPAD)

    bn = NPAD // 8
    dec = pl.pallas_call(
        _decode_body,
        grid=(NPAD // bn,),
        in_specs=[pl.BlockSpec((NC, 5, bn), lambda i: (0, 0, i))],
        out_specs=pl.BlockSpec((2, bn), lambda i: (0, i)),
        out_shape=jax.ShapeDtypeStruct((2, NPAD), jnp.float32),
    )
    return dec(parts)[:, :N_NODES].T
